# baseline (speedup 1.0000x reference)
import functools

import jax
import jax.numpy as jnp
from jax import lax
from jax.experimental import pallas as pl
from jax.experimental.pallas import tpu as pltpu
from jax.experimental.pallas import tpu_sc as plsc

_B, _S, _D = 4, 4096, 2048
_NC, _NS = 2, 16
_NW = _NC * _NS
_RPW = (_B * _S) // _NW
_WPB = _S // _RPW
_K = 8
_NBUF = 4
_NCHUNK = _RPW // _K
_NITER = _NCHUNK // _NBUF
_L = 16

_mesh = plsc.VectorSubcoreMesh(core_axis_name="c", subcore_axis_name="s")


@functools.partial(
    pl.kernel,
    mesh=_mesh,
    out_type=jax.ShapeDtypeStruct((_B * _S, _D), jnp.float32),
    scratch_types=(
        [pltpu.VMEM((_RPW,), jnp.int32)]
        + [pltpu.VMEM((_K, _D), jnp.float32)] * _NBUF
        + [pltpu.SemaphoreType.DMA] * (2 * _NBUF)
    ),
)
def _permute_rows(x_hbm, perm_hbm, out_hbm, idx_v, *rest):
    bufs = rest[:_NBUF]
    gsem = rest[_NBUF:2 * _NBUF]
    ssem = rest[2 * _NBUF:]

    wid = lax.axis_index("s") * _NC + lax.axis_index("c")
    base = wid * _RPW
    b = wid // _WPB
    soff = (wid % _WPB) * _RPW
    pltpu.sync_copy(perm_hbm.at[pl.ds(soff, _RPW)], idx_v)

    row0 = b * _S

    def add_off(k, carry):
        sl = pl.ds(k * _L, _L)
        idx_v[sl] = idx_v[sl] + row0
        return carry

    lax.fori_loop(0, _RPW // _L, add_off, 0)

    def gather(c, j):
        return pltpu.make_async_copy(
            x_hbm.at[idx_v.at[pl.ds(c * _K, _K)]], bufs[j], gsem[j])

    def store(c, j):
        return pltpu.make_async_copy(
            bufs[j], out_hbm.at[pl.ds(base + c * _K, _K)], ssem[j])

    for j in range(_NBUF):
        gather(j, j).start()

    def body(i, carry):
        for j in range(_NBUF):
            c = i * _NBUF + j
            gather(c, j).wait()
            store(c, j).start()

            @pl.when(i < _NITER - 1)
            def _():
                store(c, j).wait()
                gather(c + _NBUF, j).start()
        return carry

    lax.fori_loop(0, _NITER, body, 0)

    for j in range(_NBUF):
        store(_NCHUNK - _NBUF + j, j).wait()


def kernel(x, perm_indices):
    out = _permute_rows(x.reshape(_B * _S, _D),
                        perm_indices.astype(jnp.int32))
    return out.reshape(_B, _S, _D)
